# Initial kernel scaffold; baseline (speedup 1.0000x reference)
import jax
import jax.numpy as jnp
from jax.experimental import pallas as pl


def kernel(x, edge_index, edge_attr, batch, W_proj, b_proj, W_e1, b_e1, W_e2, b_e2, W_root, b_conv, W_gru_ih, b_gru_ih, W_gru_hh, b_gru_hh, W_r1, b_r1, W_r2, b_r2, W_p, b_p):
    raise NotImplementedError("write your pallas kernel here")


# speedup vs baseline: 3.0098x; 3.0098x over previous
import functools

import jax
import jax.numpy as jnp
from jax import lax
from jax.experimental import pallas as pl
from jax.experimental.pallas import tpu as pltpu
from jax.experimental.pallas import tpu_sc as plsc

N = 10000
E = 320000
D = 8
NG = 64
STEPS = 3

NC = 2
NS = 16
NW = NC * NS
CHUNK = 128
NCHUNKS = E // CHUNK
BASE_CHUNKS = NCHUNKS // NW
EXTRA = NCHUNKS - BASE_CHUNKS * NW
ROWS_PER_TILE = N // NS

_SC_PARAMS = pltpu.CompilerParams(use_tc_tiling_on_sc=False)


@functools.cache
def _sc_mesh():
    return plsc.VectorSubcoreMesh(
        core_axis_name="c", subcore_axis_name="s", num_cores=NC, num_subcores=NS
    )


def _gather_body(h_hbm, src_hbm, out_hbm, idx_v, rows_v, sem):
    wid = lax.axis_index("s") * NC + lax.axis_index("c")
    nchunks = BASE_CHUNKS + (wid < EXTRA).astype(jnp.int32)

    def body(jj, carry):
        g = wid + jj * NW
        pltpu.sync_copy(src_hbm.at[pl.ds(g * CHUNK, CHUNK)], idx_v)
        pltpu.async_copy(h_hbm.at[idx_v], rows_v, sem).wait()
        pltpu.sync_copy(rows_v, out_hbm.at[pl.ds(g * CHUNK, CHUNK)])
        return carry

    lax.fori_loop(0, nchunks, body, 0)


def _sc_gather(h, src):
    kern = pl.kernel(
        _gather_body,
        out_type=jax.ShapeDtypeStruct((E, D), jnp.float32),
        mesh=_sc_mesh(),
        scratch_types=[
            pltpu.VMEM((CHUNK,), jnp.int32),
            pltpu.VMEM((CHUNK, D), jnp.float32),
            pltpu.SemaphoreType.DMA,
        ],
        compiler_params=_SC_PARAMS,
    )
    return kern(h, src)


def _scatter_body(msg_hbm, dst_hbm, zeros_hbm, out_hbm, didx_v, mrows_v, agg_sh):
    cid = lax.axis_index("c")
    sid = lax.axis_index("s")
    wid = sid * NC + cid
    nchunks = BASE_CHUNKS + (wid < EXTRA).astype(jnp.int32)

    pltpu.sync_copy(
        zeros_hbm.at[pl.ds(sid * ROWS_PER_TILE, ROWS_PER_TILE)],
        agg_sh.at[pl.ds(sid * ROWS_PER_TILE, ROWS_PER_TILE)],
    )
    plsc.subcore_barrier()

    def body(jj, carry):
        g = wid + jj * NW
        pltpu.sync_copy(dst_hbm.at[pl.ds(g * CHUNK, CHUNK)], didx_v.at[0])
        pltpu.sync_copy(msg_hbm.at[pl.ds(g * CHUNK, CHUNK)], mrows_v)
        pltpu.sync_copy(mrows_v, agg_sh.at[didx_v.at[0]], add=True)
        return carry

    lax.fori_loop(0, nchunks, body, 0)
    plsc.subcore_barrier()

    pltpu.sync_copy(
        agg_sh.at[pl.ds(sid * ROWS_PER_TILE, ROWS_PER_TILE)],
        out_hbm.at[cid, pl.ds(sid * ROWS_PER_TILE, ROWS_PER_TILE)],
    )


def _sc_scatter_add(msg, dst, zeros_nd):
    kern = pl.kernel(
        _scatter_body,
        out_type=jax.ShapeDtypeStruct((NC, N, D), jnp.float32),
        mesh=_sc_mesh(),
        scratch_types=[
            pltpu.VMEM((1, CHUNK), jnp.int32),
            pltpu.VMEM((CHUNK, D), jnp.float32),
            pltpu.VMEM_SHARED((N, D), jnp.float32),
        ],
        compiler_params=_SC_PARAMS,
    )
    return kern(msg, dst, zeros_nd)


def _proj_body(x_ref, w_ref, b_ref, o_ref):
    o_ref[...] = jnp.maximum(
        jnp.dot(x_ref[...], w_ref[...], preferred_element_type=jnp.float32)
        + b_ref[...],
        0.0,
    )


def _tc_project(x, W_proj, b_proj):
    return pl.pallas_call(
        _proj_body,
        out_shape=jax.ShapeDtypeStruct((N, D), jnp.float32),
    )(x, W_proj, b_proj.reshape(1, D))


MSG_BM = 8000


def _msg_body(ea_ref, hs_ref, we1_ref, be1_ref, we2_ref, be2_ref, r_ref, s_ref, o_ref):
    eh = jnp.maximum(
        jnp.dot(ea_ref[...], we1_ref[...], preferred_element_type=jnp.float32)
        + be1_ref[...],
        0.0,
    )
    ew = (
        jnp.dot(eh, we2_ref[...], preferred_element_type=jnp.float32)
        + be2_ref[...]
    )
    hr = jnp.dot(hs_ref[...], r_ref[...], preferred_element_type=jnp.float32)
    o_ref[...] = jnp.dot(ew * hr, s_ref[...], preferred_element_type=jnp.float32)


def _tc_message(edge_attr, h_src, W_e1, b_e1, W_e2, b_e2, R, S):
    grid = E // MSG_BM
    return pl.pallas_call(
        _msg_body,
        grid=(grid,),
        in_specs=[
            pl.BlockSpec((MSG_BM, 16), lambda i: (i, 0)),
            pl.BlockSpec((MSG_BM, D), lambda i: (i, 0)),
            pl.BlockSpec((16, 16), lambda i: (0, 0)),
            pl.BlockSpec((1, 16), lambda i: (0, 0)),
            pl.BlockSpec((16, D * D), lambda i: (0, 0)),
            pl.BlockSpec((1, D * D), lambda i: (0, 0)),
            pl.BlockSpec((D, D * D), lambda i: (0, 0)),
            pl.BlockSpec((D * D, D), lambda i: (0, 0)),
        ],
        out_specs=pl.BlockSpec((MSG_BM, D), lambda i: (i, 0)),
        out_shape=jax.ShapeDtypeStruct((E, D), jnp.float32),
    )(edge_attr, h_src, W_e1, b_e1.reshape(1, 16), W_e2, b_e2.reshape(1, D * D), R, S)


def _gru_body(
    agg2_ref, h_ref, hid_ref, wroot_ref, bconv_ref,
    wir_ref, wiz_ref, win_ref, bir_ref, biz_ref, bin_ref,
    whr_ref, whz_ref, whn_ref, bhr_ref, bhz_ref, bhn_ref,
    o_ref,
):
    agg = agg2_ref[0] + agg2_ref[1]
    h = h_ref[...]
    hidden = hid_ref[...]
    m = jnp.maximum(
        agg
        + jnp.dot(h, wroot_ref[...], preferred_element_type=jnp.float32)
        + bconv_ref[...],
        0.0,
    )
    i_r = jnp.dot(m, wir_ref[...], preferred_element_type=jnp.float32) + bir_ref[...]
    i_z = jnp.dot(m, wiz_ref[...], preferred_element_type=jnp.float32) + biz_ref[...]
    i_n = jnp.dot(m, win_ref[...], preferred_element_type=jnp.float32) + bin_ref[...]
    h_r = jnp.dot(hidden, whr_ref[...], preferred_element_type=jnp.float32) + bhr_ref[...]
    h_z = jnp.dot(hidden, whz_ref[...], preferred_element_type=jnp.float32) + bhz_ref[...]
    h_n = jnp.dot(hidden, whn_ref[...], preferred_element_type=jnp.float32) + bhn_ref[...]
    r = jax.nn.sigmoid(i_r + h_r)
    z = jax.nn.sigmoid(i_z + h_z)
    n = jnp.tanh(i_n + r * h_n)
    o_ref[...] = (1.0 - z) * n + z * hidden


def _tc_gru(agg2, h, hidden, W_root, b_conv, gru_w):
    (wir, wiz, win, bir, biz, bin_, whr, whz, whn, bhr, bhz, bhn) = gru_w
    return pl.pallas_call(
        _gru_body,
        out_shape=jax.ShapeDtypeStruct((N, D), jnp.float32),
    )(agg2, h, hidden, W_root, b_conv.reshape(1, D),
      wir, wiz, win, bir, biz, bin_, whr, whz, whn, bhr, bhz, bhn)


def _readout_body(
    h_ref, batch_ref, wr1_ref, br1_ref, wr2_ref, br2_ref, wp_ref, bp_ref, o_ref
):
    h = h_ref[...]
    nf = jnp.maximum(
        jnp.dot(h, wr1_ref[...], preferred_element_type=jnp.float32) + br1_ref[...],
        0.0,
    )
    nf = jnp.dot(nf, wr2_ref[...], preferred_element_type=jnp.float32) + br2_ref[...]
    gid = lax.broadcasted_iota(jnp.int32, (1, NG), 1)
    oh = (batch_ref[...] == gid).astype(jnp.float32)
    sums = lax.dot_general(
        oh, nf, (((0,), (0,)), ((), ())), preferred_element_type=jnp.float32
    )
    counts = lax.dot_general(
        oh,
        jnp.ones((N, 1), jnp.float32),
        (((0,), (0,)), ((), ())),
        preferred_element_type=jnp.float32,
    )
    g = sums / jnp.maximum(counts, 1.0)
    o_ref[...] = (
        jnp.dot(g, wp_ref[...], preferred_element_type=jnp.float32) + bp_ref[...]
    )


def _tc_readout(h, batch2d, W_r1, b_r1, W_r2, b_r2, W_p, b_p):
    return pl.pallas_call(
        _readout_body,
        out_shape=jax.ShapeDtypeStruct((NG, 1), jnp.float32),
    )(h, batch2d, W_r1, b_r1.reshape(1, D), W_r2, b_r2.reshape(1, D),
      W_p, b_p.reshape(1, 1))


def kernel(x, edge_index, edge_attr, batch,
           W_proj, b_proj, W_e1, b_e1, W_e2, b_e2, W_root, b_conv,
           W_gru_ih, b_gru_ih, W_gru_hh, b_gru_hh,
           W_r1, b_r1, W_r2, b_r2, W_p, b_p):
    src = edge_index[0]
    dst = edge_index[1]
    batch2d = batch.reshape(N, 1)
    zeros_nd = jnp.zeros((N, D), jnp.float32)

    i8 = jnp.arange(D)
    i64 = jnp.arange(D * D)
    R = (i64[None, :] // D == i8[:, None]).astype(jnp.float32)
    S = (i64[:, None] % D == i8[None, :]).astype(jnp.float32)

    gru_w = (
        W_gru_ih[:, 0:D], W_gru_ih[:, D:2 * D], W_gru_ih[:, 2 * D:3 * D],
        b_gru_ih[0:D].reshape(1, D), b_gru_ih[D:2 * D].reshape(1, D),
        b_gru_ih[2 * D:3 * D].reshape(1, D),
        W_gru_hh[:, 0:D], W_gru_hh[:, D:2 * D], W_gru_hh[:, 2 * D:3 * D],
        b_gru_hh[0:D].reshape(1, D), b_gru_hh[D:2 * D].reshape(1, D),
        b_gru_hh[2 * D:3 * D].reshape(1, D),
    )

    h = _tc_project(x, W_proj, b_proj)
    hidden = h
    for _ in range(STEPS):
        h_src = _sc_gather(h, src)
        msg = _tc_message(edge_attr, h_src, W_e1, b_e1, W_e2, b_e2, R, S)
        agg2 = _sc_scatter_add(msg, dst, zeros_nd)
        hidden = _tc_gru(agg2, h, hidden, W_root, b_conv, gru_w)
        h = hidden
    return _tc_readout(h, batch2d, W_r1, b_r1, W_r2, b_r2, W_p, b_p)


# speedup vs baseline: 3.9344x; 1.3072x over previous
import functools

import jax
import jax.numpy as jnp
from jax import lax
from jax.experimental import pallas as pl
from jax.experimental.pallas import tpu as pltpu
from jax.experimental.pallas import tpu_sc as plsc

N = 10000
E = 320000
D = 8
NG = 64
STEPS = 3

NC = 2
NS = 16
NW = NC * NS
EPW = E // NW
CW = 125
SUB = 8
SCW = CW * SUB
NSC = EPW // SCW
NCH = EPW // CW
ROWS_PER_TILE = N // NS

_SC_PARAMS = pltpu.CompilerParams(use_tc_tiling_on_sc=False)


@functools.cache
def _sc_mesh():
    return plsc.VectorSubcoreMesh(
        core_axis_name="c", subcore_axis_name="s", num_cores=NC, num_subcores=NS
    )


def _gather_body(h_hbm, src_hbm, out_hbm, idx_v, rows_v, gsem, ssem):
    wid = lax.axis_index("s") * NC + lax.axis_index("c")
    base = wid * EPW
    pltpu.sync_copy(src_hbm.at[wid], idx_v)

    def gathers(m, half):
        descs = []
        for b in range(SUB):
            descs.append(pltpu.async_copy(
                h_hbm.at[idx_v.at[m * SUB + b]],
                rows_v.at[half, pl.ds(b * CW, CW)],
                gsem,
            ))
        return descs

    def store_desc(m, half):
        return pltpu.make_async_copy(
            rows_v.at[half], out_hbm.at[pl.ds(base + m * SCW, SCW)], ssem
        )

    def body(m, carry):
        half = lax.rem(m, 2)

        @pl.when(m >= 2)
        def _():
            store_desc(m - 2, half).wait()

        descs = gathers(m, half)
        for dsc in descs:
            dsc.wait()
        pltpu.async_copy(
            rows_v.at[half], out_hbm.at[pl.ds(base + m * SCW, SCW)], ssem
        )
        return carry

    lax.fori_loop(0, NSC, body, 0)
    store_desc(NSC - 2, lax.rem(NSC - 2, 2)).wait()
    store_desc(NSC - 1, lax.rem(NSC - 1, 2)).wait()


def _sc_gather(h, src3):
    kern = pl.kernel(
        _gather_body,
        out_type=jax.ShapeDtypeStruct((E, D), jnp.float32),
        mesh=_sc_mesh(),
        scratch_types=[
            pltpu.VMEM((NCH, CW), jnp.int32),
            pltpu.VMEM((2, SCW, D), jnp.float32),
            pltpu.SemaphoreType.DMA,
            pltpu.SemaphoreType.DMA,
        ],
        compiler_params=_SC_PARAMS,
    )
    return kern(h, src3)


def _scatter_body(msg_hbm, dst_hbm, zeros_hbm, out_hbm, didx_v, mrows_v, agg_sh,
                  lsem, asem):
    cid = lax.axis_index("c")
    sid = lax.axis_index("s")
    wid = sid * NC + cid
    base = wid * EPW

    pltpu.sync_copy(
        zeros_hbm.at[pl.ds(sid * ROWS_PER_TILE, ROWS_PER_TILE)],
        agg_sh.at[pl.ds(sid * ROWS_PER_TILE, ROWS_PER_TILE)],
    )
    pltpu.sync_copy(dst_hbm.at[wid], didx_v)
    plsc.subcore_barrier()

    def load_desc(m, half):
        return pltpu.make_async_copy(
            msg_hbm.at[pl.ds(base + m * SCW, SCW)], mrows_v.at[half], lsem
        )

    def scat_desc(m, half, b):
        return pltpu.make_async_copy(
            mrows_v.at[half, pl.ds(b * CW, CW)],
            agg_sh.at[didx_v.at[m * SUB + b]],
            asem,
        )

    def body(m, carry):
        half = lax.rem(m, 2)

        @pl.when(m >= 2)
        def _():
            for b in range(SUB):
                scat_desc(m - 2, half, b).wait()

        load_desc(m, half).start()
        load_desc(m, half).wait()
        for b in range(SUB):
            pltpu.async_copy(
                mrows_v.at[half, pl.ds(b * CW, CW)],
                agg_sh.at[didx_v.at[m * SUB + b]],
                asem,
                add=True,
            )
        return carry

    lax.fori_loop(0, NSC, body, 0)
    for m in (NSC - 2, NSC - 1):
        for b in range(SUB):
            scat_desc(m, m % 2, b).wait()
    plsc.subcore_barrier()

    pltpu.sync_copy(
        agg_sh.at[pl.ds(sid * ROWS_PER_TILE, ROWS_PER_TILE)],
        out_hbm.at[cid, pl.ds(sid * ROWS_PER_TILE, ROWS_PER_TILE)],
    )


def _sc_scatter_add(msg, dst3, zeros_nd):
    kern = pl.kernel(
        _scatter_body,
        out_type=jax.ShapeDtypeStruct((NC, N, D), jnp.float32),
        mesh=_sc_mesh(),
        scratch_types=[
            pltpu.VMEM((NCH, CW), jnp.int32),
            pltpu.VMEM((2, SCW, D), jnp.float32),
            pltpu.VMEM_SHARED((N, D), jnp.float32),
            pltpu.SemaphoreType.DMA,
            pltpu.SemaphoreType.DMA,
        ],
        compiler_params=_SC_PARAMS,
    )
    return kern(msg, dst3, zeros_nd)


def _proj_body(x_ref, w_ref, b_ref, o_ref):
    o_ref[...] = jnp.maximum(
        jnp.dot(x_ref[...], w_ref[...], preferred_element_type=jnp.float32)
        + b_ref[...],
        0.0,
    )


def _tc_project(x, W_proj, b_proj):
    return pl.pallas_call(
        _proj_body,
        out_shape=jax.ShapeDtypeStruct((N, D), jnp.float32),
    )(x, W_proj, b_proj.reshape(1, D))


MSG_BM = 8000


def _msg_body(ea_ref, hs_ref, we1_ref, be1_ref, we2_ref, be2_ref, r_ref, s_ref, o_ref):
    eh = jnp.maximum(
        jnp.dot(ea_ref[...], we1_ref[...], preferred_element_type=jnp.float32)
        + be1_ref[...],
        0.0,
    )
    ew = (
        jnp.dot(eh, we2_ref[...], preferred_element_type=jnp.float32)
        + be2_ref[...]
    )
    hr = jnp.dot(hs_ref[...], r_ref[...], preferred_element_type=jnp.float32)
    o_ref[...] = jnp.dot(ew * hr, s_ref[...], preferred_element_type=jnp.float32)


def _tc_message(edge_attr, h_src, W_e1, b_e1, W_e2, b_e2, R, S):
    grid = E // MSG_BM
    return pl.pallas_call(
        _msg_body,
        grid=(grid,),
        in_specs=[
            pl.BlockSpec((MSG_BM, 16), lambda i: (i, 0)),
            pl.BlockSpec((MSG_BM, D), lambda i: (i, 0)),
            pl.BlockSpec((16, 16), lambda i: (0, 0)),
            pl.BlockSpec((1, 16), lambda i: (0, 0)),
            pl.BlockSpec((16, D * D), lambda i: (0, 0)),
            pl.BlockSpec((1, D * D), lambda i: (0, 0)),
            pl.BlockSpec((D, D * D), lambda i: (0, 0)),
            pl.BlockSpec((D * D, D), lambda i: (0, 0)),
        ],
        out_specs=pl.BlockSpec((MSG_BM, D), lambda i: (i, 0)),
        out_shape=jax.ShapeDtypeStruct((E, D), jnp.float32),
    )(edge_attr, h_src, W_e1, b_e1.reshape(1, 16), W_e2, b_e2.reshape(1, D * D), R, S)


def _gru_body(
    agg2_ref, h_ref, hid_ref, wroot_ref, bconv_ref,
    wir_ref, wiz_ref, win_ref, bir_ref, biz_ref, bin_ref,
    whr_ref, whz_ref, whn_ref, bhr_ref, bhz_ref, bhn_ref,
    o_ref,
):
    agg = agg2_ref[0] + agg2_ref[1]
    h = h_ref[...]
    hidden = hid_ref[...]
    m = jnp.maximum(
        agg
        + jnp.dot(h, wroot_ref[...], preferred_element_type=jnp.float32)
        + bconv_ref[...],
        0.0,
    )
    i_r = jnp.dot(m, wir_ref[...], preferred_element_type=jnp.float32) + bir_ref[...]
    i_z = jnp.dot(m, wiz_ref[...], preferred_element_type=jnp.float32) + biz_ref[...]
    i_n = jnp.dot(m, win_ref[...], preferred_element_type=jnp.float32) + bin_ref[...]
    h_r = jnp.dot(hidden, whr_ref[...], preferred_element_type=jnp.float32) + bhr_ref[...]
    h_z = jnp.dot(hidden, whz_ref[...], preferred_element_type=jnp.float32) + bhz_ref[...]
    h_n = jnp.dot(hidden, whn_ref[...], preferred_element_type=jnp.float32) + bhn_ref[...]
    r = jax.nn.sigmoid(i_r + h_r)
    z = jax.nn.sigmoid(i_z + h_z)
    n = jnp.tanh(i_n + r * h_n)
    o_ref[...] = (1.0 - z) * n + z * hidden


def _tc_gru(agg2, h, hidden, W_root, b_conv, gru_w):
    (wir, wiz, win, bir, biz, bin_, whr, whz, whn, bhr, bhz, bhn) = gru_w
    return pl.pallas_call(
        _gru_body,
        out_shape=jax.ShapeDtypeStruct((N, D), jnp.float32),
    )(agg2, h, hidden, W_root, b_conv.reshape(1, D),
      wir, wiz, win, bir, biz, bin_, whr, whz, whn, bhr, bhz, bhn)


def _readout_body(
    h_ref, batch_ref, wr1_ref, br1_ref, wr2_ref, br2_ref, wp_ref, bp_ref, o_ref
):
    h = h_ref[...]
    nf = jnp.maximum(
        jnp.dot(h, wr1_ref[...], preferred_element_type=jnp.float32) + br1_ref[...],
        0.0,
    )
    nf = jnp.dot(nf, wr2_ref[...], preferred_element_type=jnp.float32) + br2_ref[...]
    gid = lax.broadcasted_iota(jnp.int32, (1, NG), 1)
    oh = (batch_ref[...] == gid).astype(jnp.float32)
    sums = lax.dot_general(
        oh, nf, (((0,), (0,)), ((), ())), preferred_element_type=jnp.float32
    )
    counts = lax.dot_general(
        oh,
        jnp.ones((N, 1), jnp.float32),
        (((0,), (0,)), ((), ())),
        preferred_element_type=jnp.float32,
    )
    g = sums / jnp.maximum(counts, 1.0)
    o_ref[...] = (
        jnp.dot(g, wp_ref[...], preferred_element_type=jnp.float32) + bp_ref[...]
    )


def _tc_readout(h, batch2d, W_r1, b_r1, W_r2, b_r2, W_p, b_p):
    return pl.pallas_call(
        _readout_body,
        out_shape=jax.ShapeDtypeStruct((NG, 1), jnp.float32),
    )(h, batch2d, W_r1, b_r1.reshape(1, D), W_r2, b_r2.reshape(1, D),
      W_p, b_p.reshape(1, 1))


def kernel(x, edge_index, edge_attr, batch,
           W_proj, b_proj, W_e1, b_e1, W_e2, b_e2, W_root, b_conv,
           W_gru_ih, b_gru_ih, W_gru_hh, b_gru_hh,
           W_r1, b_r1, W_r2, b_r2, W_p, b_p):
    src3 = edge_index[0].reshape(NW, NCH, CW)
    dst3 = edge_index[1].reshape(NW, NCH, CW)
    batch2d = batch.reshape(N, 1)
    zeros_nd = jnp.zeros((N, D), jnp.float32)

    i8 = jnp.arange(D)
    i64 = jnp.arange(D * D)
    R = (i64[None, :] // D == i8[:, None]).astype(jnp.float32)
    S = (i64[:, None] % D == i8[None, :]).astype(jnp.float32)

    gru_w = (
        W_gru_ih[:, 0:D], W_gru_ih[:, D:2 * D], W_gru_ih[:, 2 * D:3 * D],
        b_gru_ih[0:D].reshape(1, D), b_gru_ih[D:2 * D].reshape(1, D),
        b_gru_ih[2 * D:3 * D].reshape(1, D),
        W_gru_hh[:, 0:D], W_gru_hh[:, D:2 * D], W_gru_hh[:, 2 * D:3 * D],
        b_gru_hh[0:D].reshape(1, D), b_gru_hh[D:2 * D].reshape(1, D),
        b_gru_hh[2 * D:3 * D].reshape(1, D),
    )

    h = _tc_project(x, W_proj, b_proj)
    hidden = h
    for _ in range(STEPS):
        h_src = _sc_gather(h, src3)
        msg = _tc_message(edge_attr, h_src, W_e1, b_e1, W_e2, b_e2, R, S)
        agg2 = _sc_scatter_add(msg, dst3, zeros_nd)
        hidden = _tc_gru(agg2, h, hidden, W_root, b_conv, gru_w)
        h = hidden
    return _tc_readout(h, batch2d, W_r1, b_r1, W_r2, b_r2, W_p, b_p)
